# Initial kernel scaffold; baseline (speedup 1.0000x reference)
import jax
import jax.numpy as jnp
from jax.experimental import pallas as pl


def kernel(x, edge_index, batch, W1, a1_src, a1_dst, b1, W2, a2_src, a2_dst, b2, lin_W, lin_b):
    raise NotImplementedError("write your pallas kernel here")


# speedup vs baseline: 25.1639x; 25.1639x over previous
import dataclasses
import functools

import jax
import jax.numpy as jnp
from jax import lax
from jax.experimental import pallas as pl
from jax.experimental.pallas import tpu as pltpu
from jax.experimental.pallas import tpu_sc as plsc

N = 10000
NP = 10240
E = 320000
D = 128
DC = 144
NCLS = 40

NC, NS, L = 2, 16, 16
NW = NC * NS
CH = 128
NCHUNKS = E // CH
NPW = (NCHUNKS + NW - 1) // NW


def _embed_body(x_ref, w_ref, asrc_ref, adst_ref, hext_ref, as_ref, ad_ref):
    h = jnp.dot(x_ref[...], w_ref[...], preferred_element_type=jnp.float32)
    hext_ref[:, :D] = h
    col = lax.broadcasted_iota(jnp.int32, (h.shape[0], DC - D), 1)
    hext_ref[:, D:] = (col == 0).astype(jnp.float32)
    as_ref[...] = jnp.dot(h, asrc_ref[...], preferred_element_type=jnp.float32)
    ad_ref[...] = jnp.dot(h, adst_ref[...], preferred_element_type=jnp.float32)


def _tc_embed(xin, w, asrc, adst):
    return pl.pallas_call(
        _embed_body,
        out_shape=(
            jax.ShapeDtypeStruct((N, DC), jnp.float32),
            jax.ShapeDtypeStruct((N, 1), jnp.float32),
            jax.ShapeDtypeStruct((N, 1), jnp.float32),
        ),
    )(xin, w, asrc.reshape(D, 1), adst.reshape(D, 1))


def _mid_body(acc_ref, b_ref, w_ref, asrc_ref, adst_ref,
              hext_ref, as_ref, ad_ref):
    acc = acc_ref[0, :N] + acc_ref[1, :N]
    den = acc[:, D:D + 1] + 1e-16
    hin = jnp.maximum(acc[:, :D] / den + b_ref[...], 0.0)
    h = jnp.dot(hin, w_ref[...], preferred_element_type=jnp.float32)
    hext_ref[:, :D] = h
    col = lax.broadcasted_iota(jnp.int32, (h.shape[0], DC - D), 1)
    hext_ref[:, D:] = (col == 0).astype(jnp.float32)
    as_ref[...] = jnp.dot(h, asrc_ref[...], preferred_element_type=jnp.float32)
    ad_ref[...] = jnp.dot(h, adst_ref[...], preferred_element_type=jnp.float32)


def _tc_mid(acc, b, w, asrc, adst):
    return pl.pallas_call(
        _mid_body,
        out_shape=(
            jax.ShapeDtypeStruct((N, DC), jnp.float32),
            jax.ShapeDtypeStruct((N, 1), jnp.float32),
            jax.ShapeDtypeStruct((N, 1), jnp.float32),
        ),
    )(acc, b.reshape(1, D), w, asrc.reshape(D, 1), adst.reshape(D, 1))


def _final_body(acc_ref, b_ref, batch_ref, lw_ref, lb_ref, out_ref):
    acc = acc_ref[0, :N] + acc_ref[1, :N]
    den = acc[:, D:D + 1] + 1e-16
    h = acc[:, :D] / den + b_ref[...]
    gid = lax.broadcasted_iota(jnp.int32, (64, N), 0)
    onehot = (gid == batch_ref[...]).astype(jnp.float32)
    g = jnp.dot(onehot, h, preferred_element_type=jnp.float32)
    cnt = jnp.sum(onehot, axis=1, keepdims=True)
    g = g / jnp.maximum(cnt, 1.0)
    out_ref[...] = (jnp.dot(g, lw_ref[...], preferred_element_type=jnp.float32)
                    + lb_ref[...])


def _tc_final(acc, b, batch, lw_pad, lb_pad):
    return pl.pallas_call(
        _final_body,
        out_shape=jax.ShapeDtypeStruct((64, D), jnp.float32),
    )(acc, b.reshape(1, D), batch.reshape(1, N), lw_pad, lb_pad.reshape(1, D))


def _sc_gat_body(src_hbm, dst_hbm, as_hbm, ad_hbm, hext_hbm, acc_hbm,
                 as_v, ad_v, sidx, didx, rows, acc_sp, sem):
    c_id = lax.axis_index("c")
    s_id = lax.axis_index("s")
    wid = s_id * NC + c_id

    pltpu.sync_copy(as_hbm, as_v)
    pltpu.sync_copy(ad_hbm, ad_v)

    def _zrow(i, carry):
        for k in range(DC // L):
            rows[i, pl.ds(k * L, L)] = jnp.zeros((L,), jnp.float32)
        return carry
    lax.fori_loop(0, CH, _zrow, 0)
    rpt = NP // NS
    base_r = s_id * rpt
    for t in range(rpt // CH):
        pltpu.sync_copy(rows, acc_sp.at[pl.ds(base_r + t * CH, CH)])
    plsc.subcore_barrier()

    def _chunk(j, carry):
        c = wid + NW * j

        @pl.when(c < NCHUNKS)
        def _():
            off = c * CH
            pltpu.sync_copy(src_hbm.at[pl.ds(off, CH)], sidx)
            pltpu.sync_copy(dst_hbm.at[pl.ds(off, CH)], didx)
            pltpu.async_copy(hext_hbm.at[sidx], rows, sem).wait()

            def _grp(g, carry2):
                sv = sidx[pl.ds(g * L, L)]
                dv = didx[pl.ds(g * L, L)]
                e = plsc.load_gather(as_v, [sv]) + plsc.load_gather(ad_v, [dv])
                e = jnp.where(e >= 0.0, e, e * 0.2)
                w = jnp.exp(e)
                for j in range(L):
                    wj = w[j]
                    i = g * L + j
                    for k in range(DC // L):
                        sl = pl.ds(k * L, L)
                        rows[i, sl] = rows[i, sl] * wj
                return carry2
            lax.fori_loop(0, CH // L, _grp, 0)

            pltpu.sync_copy(rows, acc_sp.at[didx], add=True)
        return carry
    lax.fori_loop(0, NPW, _chunk, 0)

    plsc.subcore_barrier()
    pltpu.sync_copy(acc_sp.at[pl.ds(base_r, rpt)],
                    acc_hbm.at[c_id, pl.ds(base_r, rpt)])


def _sc_gat(src, dst, asv, adv, hext):
    mesh = plsc.VectorSubcoreMesh(core_axis_name="c", subcore_axis_name="s")
    cp = pltpu.CompilerParams(use_tc_tiling_on_sc=False)
    if "needs_layout_passes" in pltpu.CompilerParams.__dataclass_fields__:
        cp = dataclasses.replace(cp, needs_layout_passes=False)
    f = functools.partial(
        pl.kernel,
        compiler_params=cp,
        out_type=jax.ShapeDtypeStruct((NC, NP, DC), jnp.float32),
        mesh=mesh,
        scratch_types=[
            pltpu.VMEM((N,), jnp.float32),
            pltpu.VMEM((N,), jnp.float32),
            pltpu.VMEM((CH,), jnp.int32),
            pltpu.VMEM((CH,), jnp.int32),
            pltpu.VMEM((CH, DC), jnp.float32),
            pltpu.VMEM_SHARED((NP, DC), jnp.float32),
            pltpu.SemaphoreType.DMA,
        ],
    )(_sc_gat_body)
    return f(src, dst, asv, adv, hext)


def kernel(x, edge_index, batch, W1, a1_src, a1_dst, b1,
           W2, a2_src, a2_dst, b2, lin_W, lin_b):
    src = edge_index[0].astype(jnp.int32)
    dst = edge_index[1].astype(jnp.int32)
    batch32 = batch.astype(jnp.int32)

    hext1, as1, ad1 = _tc_embed(x, W1, a1_src, a1_dst)
    acc1 = _sc_gat(src, dst, as1.reshape(N), ad1.reshape(N), hext1)
    hext2, as2, ad2 = _tc_mid(acc1, b1, W2, a2_src, a2_dst)
    acc2 = _sc_gat(src, dst, as2.reshape(N), ad2.reshape(N), hext2)

    lw_pad = jnp.zeros((D, D), jnp.float32).at[:, :NCLS].set(lin_W)
    lb_pad = jnp.zeros((D,), jnp.float32).at[:NCLS].set(lin_b)
    out = _tc_final(acc2, b2, batch32, lw_pad, lb_pad)
    return out[:, :NCLS]
